# Initial kernel scaffold; baseline (speedup 1.0000x reference)
import jax
import jax.numpy as jnp
from jax.experimental import pallas as pl


def kernel(x, T):
    raise NotImplementedError("write your pallas kernel here")


# speedup vs baseline: 10.7648x; 10.7648x over previous
import functools

import jax
import jax.numpy as jnp
from jax.experimental import pallas as pl
from jax.experimental.pallas import tpu as pltpu

_ITERS = 30


def _pool_body(t_ref, x_ref, o_ref, *, k, iters):
    xb = x_ref[...]
    maxv = jnp.max(xb, axis=1, keepdims=True)
    minv = jnp.min(xb, axis=1, keepdims=True)
    kf = jnp.float32(k)

    def step(_, carry):
        lo, hi = carry
        mid = 0.5 * (lo + hi)
        cnt = jnp.sum(jnp.where(xb >= mid, 1.0, 0.0), axis=1, keepdims=True)
        ok = cnt >= kf
        return jnp.where(ok, mid, lo), jnp.where(ok, hi, mid)

    lo, _ = jax.lax.fori_loop(0, iters, step, (minv, maxv))
    t = lo
    ge = xb >= t
    cnt_ge = jnp.sum(jnp.where(ge, 1.0, 0.0), axis=1, keepdims=True)
    sum_ge = jnp.sum(jnp.where(ge, xb, 0.0), axis=1, keepdims=True)
    topk_sum = sum_ge - (cnt_ge - kf) * t
    avg = topk_sum / kf

    denom = maxv + 1e-6
    s = jnp.minimum(t / denom, maxv / denom)
    ts = jax.nn.sigmoid(t_ref[0, 0])
    logits = (s - ts) / 0.1
    gate_soft = jax.nn.sigmoid(logits)
    gate_hard = (logits >= 0).astype(jnp.float32)
    gate = (gate_hard - gate_soft) + gate_soft
    o_ref[...] = gate * maxv + (1.0 - gate) * avg


def kernel(x, T):
    B, C, H, W = x.shape
    n = H * W
    k = max(1, int(n * 0.1))
    rows = B * C
    r_blk = 8
    assert rows % r_blk == 0
    xr = x.reshape(rows, n)
    t2 = jnp.reshape(T, (1, 1)).astype(jnp.float32)

    out = pl.pallas_call(
        functools.partial(_pool_body, k=k, iters=_ITERS),
        grid=(rows // r_blk,),
        in_specs=[
            pl.BlockSpec(memory_space=pltpu.SMEM),
            pl.BlockSpec((r_blk, n), lambda i: (i, 0)),
        ],
        out_specs=pl.BlockSpec((r_blk, 1), lambda i: (i, 0)),
        out_shape=jax.ShapeDtypeStruct((rows, 1), jnp.float32),
        compiler_params=pltpu.CompilerParams(
            dimension_semantics=("arbitrary",),
        ),
    )(t2, xr)
    return out.reshape(B, C)


# speedup vs baseline: 18.7790x; 1.7445x over previous
import functools

import jax
import jax.numpy as jnp
from jax.experimental import pallas as pl
from jax.experimental.pallas import tpu as pltpu

_ITERS = 26
_SPLIT = 8


def _split_reduce(arr, op, n):
    step = n // _SPLIT
    parts = [
        op(arr[:, j * step:(j + 1) * step], axis=1, keepdims=True)
        for j in range(_SPLIT)
    ]
    while len(parts) > 1:
        parts = [
            (parts[i] + parts[i + 1]) if op is jnp.sum
            else op(jnp.concatenate([parts[i], parts[i + 1]], axis=1),
                    axis=1, keepdims=True)
            for i in range(0, len(parts), 2)
        ]
    return parts[0]


def _pool_body(t_ref, x_ref, o_ref, *, k, n, iters):
    xb = x_ref[...]
    maxv = _split_reduce(xb, jnp.max, n)
    minv = _split_reduce(xb, jnp.min, n)
    kf = jnp.float32(k)

    def step(_, carry):
        lo, hi = carry
        mid = 0.5 * (lo + hi)
        cnt = _split_reduce(jnp.where(xb >= mid, 1.0, 0.0), jnp.sum, n)
        ok = cnt >= kf
        return jnp.where(ok, mid, lo), jnp.where(ok, hi, mid)

    lo, _ = jax.lax.fori_loop(0, iters, step, (minv, maxv))
    t = lo
    ge = xb >= t
    cnt_ge = _split_reduce(jnp.where(ge, 1.0, 0.0), jnp.sum, n)
    sum_ge = _split_reduce(jnp.where(ge, xb, 0.0), jnp.sum, n)
    topk_sum = sum_ge - (cnt_ge - kf) * t
    avg = topk_sum / kf

    denom = maxv + 1e-6
    s = jnp.minimum(t / denom, maxv / denom)
    ts = jax.nn.sigmoid(t_ref[0, 0])
    logits = (s - ts) / 0.1
    gate_soft = jax.nn.sigmoid(logits)
    gate_hard = (logits >= 0).astype(jnp.float32)
    gate = (gate_hard - gate_soft) + gate_soft
    o_ref[...] = gate * maxv + (1.0 - gate) * avg


def kernel(x, T):
    B, C, H, W = x.shape
    n = H * W
    k = max(1, int(n * 0.1))
    rows = B * C
    r_blk = 8
    assert rows % r_blk == 0 and n % (_SPLIT * 128) == 0
    xr = x.reshape(rows, n)
    t2 = jnp.reshape(T, (1, 1)).astype(jnp.float32)

    out = pl.pallas_call(
        functools.partial(_pool_body, k=k, n=n, iters=_ITERS),
        grid=(rows // r_blk,),
        in_specs=[
            pl.BlockSpec(memory_space=pltpu.SMEM),
            pl.BlockSpec((r_blk, n), lambda i: (i, 0)),
        ],
        out_specs=pl.BlockSpec((r_blk, 1), lambda i: (i, 0)),
        out_shape=jax.ShapeDtypeStruct((rows, 1), jnp.float32),
        compiler_params=pltpu.CompilerParams(
            dimension_semantics=("arbitrary",),
        ),
    )(t2, xr)
    return out.reshape(B, C)


# speedup vs baseline: 24.8621x; 1.3239x over previous
import functools

import jax
import jax.numpy as jnp
from jax.experimental import pallas as pl
from jax.experimental.pallas import tpu as pltpu

_ITERS = 18
_SPLIT = 14


def _split_reduce(arr, op, combine, n):
    step = n // _SPLIT
    parts = [
        op(arr[:, j * step:(j + 1) * step], axis=1, keepdims=True)
        for j in range(_SPLIT)
    ]
    while len(parts) > 1:
        nxt = [combine(parts[i], parts[i + 1])
               for i in range(0, len(parts) - 1, 2)]
        if len(parts) % 2:
            nxt.append(parts[-1])
        parts = nxt
    return parts[0]


def _pool_body(t_ref, x_ref, o_ref, *, k, n, iters):
    xb = x_ref[...]
    maxv = _split_reduce(xb, jnp.max, jnp.maximum, n)
    minv = _split_reduce(xb, jnp.min, jnp.minimum, n)
    kf = jnp.float32(k)

    def step(_, carry):
        lo, hi = carry
        mid = 0.5 * (lo + hi)
        cnt = _split_reduce(jnp.where(xb >= mid, 1.0, 0.0), jnp.sum, jnp.add, n)
        ok = cnt >= kf
        return jnp.where(ok, mid, lo), jnp.where(ok, hi, mid)

    lo, _ = jax.lax.fori_loop(0, iters, step, (minv, maxv))
    t = lo
    ge = xb >= t
    cnt_ge = _split_reduce(jnp.where(ge, 1.0, 0.0), jnp.sum, jnp.add, n)
    sum_ge = _split_reduce(jnp.where(ge, xb, 0.0), jnp.sum, jnp.add, n)
    topk_sum = sum_ge - (cnt_ge - kf) * t
    avg = topk_sum / kf

    denom = maxv + 1e-6
    s = jnp.minimum(t / denom, maxv / denom)
    ts = jax.nn.sigmoid(t_ref[0, 0])
    logits = (s - ts) / 0.1
    gate_soft = jax.nn.sigmoid(logits)
    gate_hard = (logits >= 0).astype(jnp.float32)
    gate = (gate_hard - gate_soft) + gate_soft
    o_ref[...] = gate * maxv + (1.0 - gate) * avg


def kernel(x, T):
    B, C, H, W = x.shape
    n = H * W
    k = max(1, int(n * 0.1))
    rows = B * C
    r_blk = 8
    assert rows % r_blk == 0 and (n // 128) % _SPLIT == 0
    xr = x.reshape(rows, n)
    t2 = jnp.reshape(T, (1, 1)).astype(jnp.float32)

    out = pl.pallas_call(
        functools.partial(_pool_body, k=k, n=n, iters=_ITERS),
        grid=(rows // r_blk,),
        in_specs=[
            pl.BlockSpec(memory_space=pltpu.SMEM),
            pl.BlockSpec((r_blk, n), lambda i: (i, 0)),
        ],
        out_specs=pl.BlockSpec((r_blk, 1), lambda i: (i, 0)),
        out_shape=jax.ShapeDtypeStruct((rows, 1), jnp.float32),
        compiler_params=pltpu.CompilerParams(
            dimension_semantics=("arbitrary",),
        ),
    )(t2, xr)
    return out.reshape(B, C)


# speedup vs baseline: 28.1926x; 1.1340x over previous
import functools

import jax
import jax.numpy as jnp
from jax.experimental import pallas as pl
from jax.experimental.pallas import tpu as pltpu

_ITERS = 18
_SPLIT = 14


def _split_reduce(arr, op, combine, n):
    step = n // _SPLIT
    parts = [
        op(arr[:, j * step:(j + 1) * step], axis=1, keepdims=True)
        for j in range(_SPLIT)
    ]
    while len(parts) > 1:
        nxt = [combine(parts[i], parts[i + 1])
               for i in range(0, len(parts) - 1, 2)]
        if len(parts) % 2:
            nxt.append(parts[-1])
        parts = nxt
    return parts[0]


def _pool_body(t_ref, x_ref, o_ref, *, k, n, iters):
    xb = x_ref[...]
    maxv = _split_reduce(xb, jnp.max, jnp.maximum, n)
    minv = _split_reduce(xb, jnp.min, jnp.minimum, n)
    kf = jnp.float32(k)

    def step(_, carry):
        lo, hi = carry
        mid = 0.5 * (lo + hi)
        cnt = _split_reduce(jnp.where(xb >= mid, 1.0, 0.0), jnp.sum, jnp.add, n)
        ok = cnt >= kf
        return jnp.where(ok, mid, lo), jnp.where(ok, hi, mid)

    lo, _ = jax.lax.fori_loop(0, iters, step, (minv, maxv))
    t = lo
    ge = xb >= t
    cnt_ge = _split_reduce(jnp.where(ge, 1.0, 0.0), jnp.sum, jnp.add, n)
    sum_ge = _split_reduce(jnp.where(ge, xb, 0.0), jnp.sum, jnp.add, n)
    topk_sum = sum_ge - (cnt_ge - kf) * t
    avg = topk_sum / kf

    denom = maxv + 1e-6
    s = jnp.minimum(t / denom, maxv / denom)
    ts = jax.nn.sigmoid(t_ref[0, 0])
    logits = (s - ts) / 0.1
    gate_soft = jax.nn.sigmoid(logits)
    gate_hard = (logits >= 0).astype(jnp.float32)
    gate = (gate_hard - gate_soft) + gate_soft
    o_ref[...] = gate * maxv + (1.0 - gate) * avg


def kernel(x, T):
    B, C, H, W = x.shape
    n = H * W
    k = max(1, int(n * 0.1))
    rows = B * C
    r_blk = 16
    assert rows % r_blk == 0 and (n // 128) % _SPLIT == 0
    xr = x.reshape(rows, n)
    t2 = jnp.reshape(T, (1, 1)).astype(jnp.float32)

    out = pl.pallas_call(
        functools.partial(_pool_body, k=k, n=n, iters=_ITERS),
        grid=(rows // r_blk,),
        in_specs=[
            pl.BlockSpec(memory_space=pltpu.SMEM),
            pl.BlockSpec((r_blk, n), lambda i: (i, 0)),
        ],
        out_specs=pl.BlockSpec((r_blk, 1), lambda i: (i, 0)),
        out_shape=jax.ShapeDtypeStruct((rows, 1), jnp.float32),
        compiler_params=pltpu.CompilerParams(
            dimension_semantics=("arbitrary",),
        ),
    )(t2, xr)
    return out.reshape(B, C)


# speedup vs baseline: 30.0817x; 1.0670x over previous
import functools

import jax
import jax.numpy as jnp
from jax.experimental import pallas as pl
from jax.experimental.pallas import tpu as pltpu

_ITERS = 18
_SPLIT = 14


def _split_reduce(arr, op, combine, n):
    step = n // _SPLIT
    parts = [
        op(arr[:, j * step:(j + 1) * step], axis=1, keepdims=True)
        for j in range(_SPLIT)
    ]
    while len(parts) > 1:
        nxt = [combine(parts[i], parts[i + 1])
               for i in range(0, len(parts) - 1, 2)]
        if len(parts) % 2:
            nxt.append(parts[-1])
        parts = nxt
    return parts[0]


def _pool_body(t_ref, x_ref, o_ref, *, k, n, iters):
    xb = x_ref[...]
    maxv = _split_reduce(xb, jnp.max, jnp.maximum, n)
    minv = _split_reduce(xb, jnp.min, jnp.minimum, n)
    kf = jnp.float32(k)

    def step(_, carry):
        lo, hi = carry
        mid = 0.5 * (lo + hi)
        cnt = _split_reduce(jnp.where(xb >= mid, 1.0, 0.0), jnp.sum, jnp.add, n)
        ok = cnt >= kf
        return jnp.where(ok, mid, lo), jnp.where(ok, hi, mid)

    lo, _ = jax.lax.fori_loop(0, iters, step, (minv, maxv))
    t = lo
    ge = xb >= t
    cnt_ge = _split_reduce(jnp.where(ge, 1.0, 0.0), jnp.sum, jnp.add, n)
    sum_ge = _split_reduce(jnp.where(ge, xb, 0.0), jnp.sum, jnp.add, n)
    topk_sum = sum_ge - (cnt_ge - kf) * t
    avg = topk_sum / kf

    denom = maxv + 1e-6
    s = jnp.minimum(t / denom, maxv / denom)
    ts = jax.nn.sigmoid(t_ref[0, 0])
    logits = (s - ts) / 0.1
    gate_soft = jax.nn.sigmoid(logits)
    gate_hard = (logits >= 0).astype(jnp.float32)
    gate = (gate_hard - gate_soft) + gate_soft
    o_ref[...] = gate * maxv + (1.0 - gate) * avg


def kernel(x, T):
    B, C, H, W = x.shape
    n = H * W
    k = max(1, int(n * 0.1))
    rows = B * C
    r_blk = 32
    assert rows % r_blk == 0 and (n // 128) % _SPLIT == 0
    xr = x.reshape(rows, n)
    t2 = jnp.reshape(T, (1, 1)).astype(jnp.float32)

    out = pl.pallas_call(
        functools.partial(_pool_body, k=k, n=n, iters=_ITERS),
        grid=(rows // r_blk,),
        in_specs=[
            pl.BlockSpec(memory_space=pltpu.SMEM),
            pl.BlockSpec((r_blk, n), lambda i: (i, 0)),
        ],
        out_specs=pl.BlockSpec((r_blk, 1), lambda i: (i, 0)),
        out_shape=jax.ShapeDtypeStruct((rows, 1), jnp.float32),
        compiler_params=pltpu.CompilerParams(
            dimension_semantics=("arbitrary",),
        ),
    )(t2, xr)
    return out.reshape(B, C)


# speedup vs baseline: 33.6471x; 1.1185x over previous
import functools

import jax
import jax.numpy as jnp
from jax.experimental import pallas as pl
from jax.experimental.pallas import tpu as pltpu

_ITERS = 14
_SPLIT = 14


def _split_reduce(arr, op, combine, n):
    step = n // _SPLIT
    parts = [
        op(arr[:, j * step:(j + 1) * step], axis=1, keepdims=True)
        for j in range(_SPLIT)
    ]
    while len(parts) > 1:
        nxt = [combine(parts[i], parts[i + 1])
               for i in range(0, len(parts) - 1, 2)]
        if len(parts) % 2:
            nxt.append(parts[-1])
        parts = nxt
    return parts[0]


def _pool_body(t_ref, x_ref, o_ref, *, k, n, iters):
    xb = x_ref[...]
    maxv = _split_reduce(xb, jnp.max, jnp.maximum, n)
    minv = _split_reduce(xb, jnp.min, jnp.minimum, n)
    kf = jnp.float32(k)

    def step(_, carry):
        lo, hi = carry
        mid = 0.5 * (lo + hi)
        cnt = _split_reduce(jnp.where(xb >= mid, 1.0, 0.0), jnp.sum, jnp.add, n)
        ok = cnt >= kf
        return jnp.where(ok, mid, lo), jnp.where(ok, hi, mid)

    lo, _ = jax.lax.fori_loop(0, iters, step, (minv, maxv))
    t = lo
    ge = xb >= t
    cnt_ge = _split_reduce(jnp.where(ge, 1.0, 0.0), jnp.sum, jnp.add, n)
    sum_ge = _split_reduce(jnp.where(ge, xb, 0.0), jnp.sum, jnp.add, n)
    topk_sum = sum_ge - (cnt_ge - kf) * t
    avg = topk_sum / kf

    denom = maxv + 1e-6
    s = jnp.minimum(t / denom, maxv / denom)
    ts = jax.nn.sigmoid(t_ref[0, 0])
    logits = (s - ts) / 0.1
    gate_soft = jax.nn.sigmoid(logits)
    gate_hard = (logits >= 0).astype(jnp.float32)
    gate = (gate_hard - gate_soft) + gate_soft
    o_ref[...] = gate * maxv + (1.0 - gate) * avg


def kernel(x, T):
    B, C, H, W = x.shape
    n = H * W
    k = max(1, int(n * 0.1))
    rows = B * C
    r_blk = 32
    assert rows % r_blk == 0 and (n // 128) % _SPLIT == 0
    xr = x.reshape(rows, n)
    t2 = jnp.reshape(T, (1, 1)).astype(jnp.float32)

    out = pl.pallas_call(
        functools.partial(_pool_body, k=k, n=n, iters=_ITERS),
        grid=(rows // r_blk,),
        in_specs=[
            pl.BlockSpec(memory_space=pltpu.SMEM),
            pl.BlockSpec((r_blk, n), lambda i: (i, 0)),
        ],
        out_specs=pl.BlockSpec((r_blk, 1), lambda i: (i, 0)),
        out_shape=jax.ShapeDtypeStruct((rows, 1), jnp.float32),
        compiler_params=pltpu.CompilerParams(
            dimension_semantics=("arbitrary",),
        ),
    )(t2, xr)
    return out.reshape(B, C)
